# baseline (speedup 1.0000x reference)
import functools

import jax
import jax.numpy as jnp
from jax import lax
from jax.experimental import pallas as pl
from jax.experimental.pallas import tpu as pltpu
from jax.experimental.pallas import tpu_sc as plsc

_NC = 2
_NS = 16
_NW = _NC * _NS
_LANES = 16


def _make_sc_kernel(B, S, D):
    SPW = S // _NW
    CS = 32
    CHUNK_W = CS * D
    NVREG = CHUNK_W // _LANES
    UNROLL = 8

    mesh = plsc.VectorSubcoreMesh(core_axis_name="c", subcore_axis_name="s")

    @functools.partial(
        pl.kernel,
        mesh=mesh,
        out_type=jax.ShapeDtypeStruct((B * S * D,), jnp.float32),
        scratch_types=[
            pltpu.VMEM((CHUNK_W,), jnp.float32),
            pltpu.VMEM((CHUNK_W,), jnp.float32),
        ],
    )
    def sc_kernel(x_hbm, pe_hbm, out_hbm, pe_v, x_v):
        wid = lax.axis_index("s") * _NC + lax.axis_index("c")
        s0 = wid * SPW
        for k in range(SPW // CS):
            pe_off = (s0 + k * CS) * D
            pltpu.sync_copy(pe_hbm.at[pl.ds(pe_off, CHUNK_W)], pe_v)
            for b in range(B):
                x_off = b * S * D + pe_off
                pltpu.sync_copy(x_hbm.at[pl.ds(x_off, CHUNK_W)], x_v)

                def add_body(i, _):
                    base = i * (_LANES * UNROLL)
                    for u in range(UNROLL):
                        o = base + u * _LANES
                        x_v[pl.ds(o, _LANES)] = (
                            x_v[pl.ds(o, _LANES)] + pe_v[pl.ds(o, _LANES)]
                        )
                    return 0

                lax.fori_loop(0, NVREG // UNROLL, add_body, 0)
                pltpu.sync_copy(x_v, out_hbm.at[pl.ds(x_off, CHUNK_W)])

    return sc_kernel


def kernel(x, pos_emb):
    B, S, D = x.shape
    pe = pos_emb[:S]
    sc = _make_sc_kernel(B, S, D)
    out = sc(x.reshape(-1), pe.reshape(-1))
    return out.reshape(B, S, D)


# speedup vs baseline: 1.1896x; 1.1896x over previous
import functools

import jax
import jax.numpy as jnp
from jax import lax
from jax.experimental import pallas as pl
from jax.experimental.pallas import tpu as pltpu
from jax.experimental.pallas import tpu_sc as plsc

_NC = 2
_NS = 16
_NW = _NC * _NS
_LANES = 16


def _make_sc_kernel(B, S, D):
    SPW = S // _NW
    CS = 16
    CHUNK_W = CS * D
    NVREG = CHUNK_W // _LANES
    NCH = SPW // CS
    NSTEP = NCH * B

    mesh = plsc.VectorSubcoreMesh(core_axis_name="c", subcore_axis_name="s")

    @functools.partial(
        pl.kernel,
        mesh=mesh,
        out_type=jax.ShapeDtypeStruct((B * S * D,), jnp.float32),
        scratch_types=[
            pltpu.VMEM((CHUNK_W,), jnp.float32),
            pltpu.VMEM((CHUNK_W,), jnp.float32),
            pltpu.VMEM((CHUNK_W,), jnp.float32),
            pltpu.VMEM((CHUNK_W,), jnp.float32),
            pltpu.VMEM((CHUNK_W,), jnp.float32),
            pltpu.VMEM((CHUNK_W,), jnp.float32),
            pltpu.SemaphoreType.DMA,
            pltpu.SemaphoreType.DMA,
            pltpu.SemaphoreType.DMA,
            pltpu.SemaphoreType.DMA,
            pltpu.SemaphoreType.DMA,
            pltpu.SemaphoreType.DMA,
            pltpu.SemaphoreType.DMA,
            pltpu.SemaphoreType.DMA,
            pltpu.SemaphoreType.DMA,
            pltpu.SemaphoreType.DMA,
        ],
    )
    def sc_kernel(x_hbm, pe_hbm, out_hbm,
                  xa, xb, xc, xd, pa, pb,
                  sia, sib, sic, sid, soa, sob, soc, sod, spa, spb):
        xbuf = [xa, xb, xc, xd]
        pbuf = [pa, pb]
        sin = [sia, sib, sic, sid]
        sout = [soa, sob, soc, sod]
        spe = [spa, spb]

        wid = lax.axis_index("s") * _NC + lax.axis_index("c")
        s0 = wid * SPW

        def x_off(t):
            k, b = divmod(t, B)
            return b * S * D + (s0 + k * CS) * D

        in_h = [None] * NSTEP
        out_h = [None] * NSTEP
        pe_h = [None] * NCH

        def start_in(t):
            j = t % 4
            in_h[t] = pltpu.async_copy(
                x_hbm.at[pl.ds(x_off(t), CHUNK_W)], xbuf[j], sin[j])

        def start_pe(k):
            pe_h[k] = pltpu.async_copy(
                pe_hbm.at[pl.ds((s0 + k * CS) * D, CHUNK_W)], pbuf[k % 2],
                spe[k % 2])

        start_pe(0)
        start_in(0)
        start_in(1)
        for t in range(NSTEP):
            k, b = divmod(t, B)
            j = t % 4
            if b == 0:
                if k + 1 < NCH:
                    start_pe(k + 1)
                pe_h[k].wait()
            in_h[t].wait()

            src = xbuf[j]
            pe = pbuf[k % 2]

            @plsc.parallel_loop(0, NVREG, 1, unroll=8)
            def _add(i):
                o = i * _LANES
                src[pl.ds(o, _LANES)] = (
                    src[pl.ds(o, _LANES)] + pe[pl.ds(o, _LANES)])

            out_h[t] = pltpu.async_copy(
                xbuf[j], out_hbm.at[pl.ds(x_off(t), CHUNK_W)], sout[j])
            if t >= 2:
                out_h[t - 2].wait()
            if t + 2 < NSTEP:
                start_in(t + 2)
        out_h[NSTEP - 2].wait()
        out_h[NSTEP - 1].wait()

    return sc_kernel


def kernel(x, pos_emb):
    B, S, D = x.shape
    pe = pos_emb[:S]
    sc = _make_sc_kernel(B, S, D)
    out = sc(x.reshape(-1), pe.reshape(-1))
    return out.reshape(B, S, D)


# speedup vs baseline: 3.2049x; 2.6942x over previous
import functools

import jax
import jax.numpy as jnp
from jax import lax
from jax.experimental import pallas as pl
from jax.experimental.pallas import tpu as pltpu
from jax.experimental.pallas import tpu_sc as plsc

_NC = 2
_NS = 16
_NW = _NC * _NS
_LANES = 16


def _make_sc_kernel(B, S, D):
    SPW = S // _NW
    CS = 16
    NVREG = CS * D // _LANES
    VPR = D // _LANES
    NCH = SPW // CS
    NSTEP = NCH * B

    mesh = plsc.VectorSubcoreMesh(core_axis_name="c", subcore_axis_name="s")

    @functools.partial(
        pl.kernel,
        mesh=mesh,
        out_type=jax.ShapeDtypeStruct((B, S, D), jnp.float32),
        scratch_types=[
            pltpu.VMEM((CS, D), jnp.float32),
            pltpu.VMEM((CS, D), jnp.float32),
            pltpu.VMEM((CS, D), jnp.float32),
            pltpu.VMEM((CS, D), jnp.float32),
            pltpu.VMEM((CS, D), jnp.float32),
            pltpu.VMEM((CS, D), jnp.float32),
            pltpu.SemaphoreType.DMA,
            pltpu.SemaphoreType.DMA,
            pltpu.SemaphoreType.DMA,
            pltpu.SemaphoreType.DMA,
            pltpu.SemaphoreType.DMA,
            pltpu.SemaphoreType.DMA,
            pltpu.SemaphoreType.DMA,
            pltpu.SemaphoreType.DMA,
            pltpu.SemaphoreType.DMA,
            pltpu.SemaphoreType.DMA,
        ],
    )
    def sc_kernel(x_hbm, pe_hbm, out_hbm,
                  xa, xb, xc, xd, pa, pb,
                  sia, sib, sic, sid, soa, sob, soc, sod, spa, spb):
        xbuf = [xa, xb, xc, xd]
        pbuf = [pa, pb]
        sin = [sia, sib, sic, sid]
        sout = [soa, sob, soc, sod]
        spe = [spa, spb]

        wid = lax.axis_index("s") * _NC + lax.axis_index("c")
        s0 = wid * SPW

        in_h = [None] * NSTEP
        out_h = [None] * NSTEP
        pe_h = [None] * NCH

        def rows(t):
            k, b = divmod(t, B)
            return k, b, s0 + k * CS

        def start_in(t):
            k, b, lo = rows(t)
            j = t % 4
            in_h[t] = pltpu.async_copy(
                x_hbm.at[b, pl.ds(lo, CS), :], xbuf[j], sin[j])

        def start_pe(k):
            pe_h[k] = pltpu.async_copy(
                pe_hbm.at[pl.ds(s0 + k * CS, CS), :], pbuf[k % 2],
                spe[k % 2])

        start_pe(0)
        start_in(0)
        start_in(1)
        for t in range(NSTEP):
            k, b, lo = rows(t)
            j = t % 4
            if b == 0:
                if k + 1 < NCH:
                    start_pe(k + 1)
                pe_h[k].wait()
            in_h[t].wait()

            src = xbuf[j]
            pe = pbuf[k % 2]

            @plsc.parallel_loop(0, NVREG, 1, unroll=8)
            def _add(i):
                r = lax.shift_right_logical(i, 6)
                c = pl.multiple_of(
                    lax.shift_left(lax.bitwise_and(i, VPR - 1), 4), _LANES)
                src[r, pl.ds(c, _LANES)] = (
                    src[r, pl.ds(c, _LANES)] + pe[r, pl.ds(c, _LANES)])

            out_h[t] = pltpu.async_copy(
                xbuf[j], out_hbm.at[b, pl.ds(lo, CS), :], sout[j])
            if t >= 2:
                out_h[t - 2].wait()
            if t + 2 < NSTEP:
                start_in(t + 2)
        out_h[NSTEP - 2].wait()
        out_h[NSTEP - 1].wait()

    return sc_kernel


def kernel(x, pos_emb):
    B, S, D = x.shape
    pe = pos_emb[:S]
    sc = _make_sc_kernel(B, S, D)
    return sc(x, pe)


# speedup vs baseline: 3.2802x; 1.0235x over previous
import functools

import jax
import jax.numpy as jnp
from jax import lax
from jax.experimental import pallas as pl
from jax.experimental.pallas import tpu as pltpu
from jax.experimental.pallas import tpu_sc as plsc

_NC = 2
_NS = 16
_NW = _NC * _NS
_LANES = 16


def _make_sc_kernel(B, S, D):
    SPW = S // _NW
    CS = 8
    NVREG = CS * D // _LANES
    VPR = D // _LANES
    NCH = SPW // CS

    mesh = plsc.VectorSubcoreMesh(core_axis_name="c", subcore_axis_name="s")

    @functools.partial(
        pl.kernel,
        mesh=mesh,
        out_type=jax.ShapeDtypeStruct((B, S, D), jnp.float32),
        scratch_types=[
            pltpu.VMEM((2, B, CS, D), jnp.float32),
            pltpu.VMEM((2, CS, D), jnp.float32),
            [pltpu.SemaphoreType.DMA] * (2 * B),
            [pltpu.SemaphoreType.DMA] * (2 * B),
            [pltpu.SemaphoreType.DMA] * 2,
        ],
    )
    def sc_kernel(x_hbm, pe_hbm, out_hbm, xbuf, pbuf, sin, sout, spe):
        wid = lax.axis_index("s") * _NC + lax.axis_index("c")
        s0 = wid * SPW

        in_h = [[None] * B for _ in range(NCH)]
        out_h = [[None] * B for _ in range(NCH)]
        pe_h = [None] * NCH

        def start_in(k):
            p = k % 2
            lo = s0 + k * CS
            for b in range(B):
                in_h[k][b] = pltpu.async_copy(
                    x_hbm.at[b, pl.ds(lo, CS), :], xbuf.at[p, b],
                    sin[p * B + b])

        def start_pe(k):
            pe_h[k] = pltpu.async_copy(
                pe_hbm.at[pl.ds(s0 + k * CS, CS), :], pbuf.at[k % 2],
                spe[k % 2])

        start_pe(0)
        start_in(0)
        start_pe(1)
        start_in(1)
        for k in range(NCH):
            p = k % 2
            lo = s0 + k * CS
            pe_h[k].wait()
            for b in range(B):
                in_h[k][b].wait()

            @plsc.parallel_loop(0, NVREG, 1, unroll=4)
            def _add(i):
                r = lax.shift_right_logical(i, 6)
                c = pl.multiple_of(
                    lax.shift_left(lax.bitwise_and(i, VPR - 1), 4), _LANES)
                pv = pbuf[p, r, pl.ds(c, _LANES)]
                for b in range(B):
                    xbuf[p, b, r, pl.ds(c, _LANES)] = (
                        xbuf[p, b, r, pl.ds(c, _LANES)] + pv)

            for b in range(B):
                out_h[k][b] = pltpu.async_copy(
                    xbuf.at[p, b], out_hbm.at[b, pl.ds(lo, CS), :],
                    sout[p * B + b])
            if k >= 1:
                for b in range(B):
                    out_h[k - 1][b].wait()
            if k + 2 < NCH:
                start_pe(k + 2)
                start_in(k + 2)
        for b in range(B):
            out_h[NCH - 1][b].wait()

    return sc_kernel


def kernel(x, pos_emb):
    B, S, D = x.shape
    pe = pos_emb[:S]
    sc = _make_sc_kernel(B, S, D)
    return sc(x, pe)
